# baseline (speedup 1.0000x reference)
import functools

import jax
import jax.numpy as jnp
from jax import lax
from jax.experimental import pallas as pl
from jax.experimental.pallas import tpu as pltpu
from jax.experimental.pallas import tpu_sc as plsc

DIM = 768
N_EXPERT = 16
MULT = 4
HID = DIM * MULT
N_TOK = 2048

BT = 128
NB = 32
NP = NB * BT
BH = 1536
NH = HID // BH

NC = 2
NS = 16
NW = NC * NS


def _router_body(x_ref, wg_ref, eid_ref, gate_ref, rank_ref, cnt_ref):
    x = x_ref[...]
    logits = jnp.dot(x, wg_ref[...], preferred_element_type=jnp.float32)
    probs = jax.nn.softmax(logits, axis=-1)
    top = jnp.argmax(probs, axis=-1)
    gate = jnp.max(probs, axis=-1)
    lane = jax.lax.broadcasted_iota(jnp.int32, (N_TOK, N_EXPERT), 1)
    onehot = jnp.where(lane == top[:, None], 1.0, 0.0)
    ii = jax.lax.broadcasted_iota(jnp.int32, (N_TOK, N_TOK), 0)
    jj = jax.lax.broadcasted_iota(jnp.int32, (N_TOK, N_TOK), 1)
    ltri = jnp.where(ii > jj, 1.0, 0.0)
    rk = jnp.dot(ltri, onehot, preferred_element_type=jnp.float32)
    rank = jnp.sum(rk * onehot, axis=1)
    eid_ref[...] = top[:, None].astype(jnp.int32)
    gate_ref[...] = gate[:, None]
    rank_ref[...] = rank[:, None].astype(jnp.int32)
    cnt_ref[...] = jnp.sum(onehot, axis=0, keepdims=True).astype(jnp.int32)


def _router(x, Wg):
    return pl.pallas_call(
        _router_body,
        out_shape=[
            jax.ShapeDtypeStruct((N_TOK, 1), jnp.int32),
            jax.ShapeDtypeStruct((N_TOK, 1), jnp.float32),
            jax.ShapeDtypeStruct((N_TOK, 1), jnp.int32),
            jax.ShapeDtypeStruct((1, N_EXPERT), jnp.int32),
        ],
    )(x, Wg)


def _dispatch_body(eid_hbm, rank_hbm, gate_hbm, cnt_hbm, x_hbm,
                   dest_hbm, gsort_hbm, be_hbm, xs_hbm,
                   eid_v, rank_v, gate_v, cnt_v, pad_v,
                   dest_v, src_v, gsort_v, be_v,
                   src_sh, idx_v, rows_v, sem, idx_b, rows_b, sem_b):
    c = lax.axis_index("c")
    s = lax.axis_index("s")

    @pl.when(s == 0)
    def _work():
        pltpu.sync_copy(eid_hbm, eid_v)
        pltpu.sync_copy(rank_hbm, rank_v)
        pltpu.sync_copy(gate_hbm, gate_v)
        pltpu.sync_copy(cnt_hbm, cnt_v)

        cnt = cnt_v[...]
        nblk = (cnt + (BT - 1)) >> 7
        incl = plsc.cumsum(nblk)
        excl = incl - nblk
        pad_v[...] = excl * BT
        iota16 = lax.iota(jnp.int32, 16)
        last_e = jnp.max(jnp.where(nblk > 0, iota16, 0))

        for k in range(NB // 16):
            bvec = iota16 + 16 * k
            be_raw = jnp.zeros((16,), jnp.int32)
            for e in range(N_EXPERT):
                incl_e = jnp.sum(jnp.where(iota16 == e, incl, 0))
                be_raw = be_raw + jnp.where(incl_e <= bvec, 1, 0)
            be_v[pl.ds(16 * k, 16)] = jnp.minimum(be_raw, last_e)

        zeros_f = jnp.zeros((16,), jnp.float32)

        def _zfill(i, carry):
            src_v[pl.ds(i * 16, 16)] = (iota16 + i * 16) & (N_TOK - 1)
            gsort_v[pl.ds(i * 16, 16)] = zeros_f
            return carry

        lax.fori_loop(0, NP // 16, _zfill, 0)

        def _scat(i, carry):
            base = i * 16
            ev = eid_v[pl.ds(base, 16)]
            rv = rank_v[pl.ds(base, 16)]
            gv = gate_v[pl.ds(base, 16)]
            po = plsc.load_gather(pad_v, [ev])
            dv = po + rv
            dest_v[pl.ds(base, 16)] = dv
            plsc.store_scatter(src_v, [dv], iota16 + base)
            plsc.store_scatter(gsort_v, [dv], gv)
            return carry

        lax.fori_loop(0, N_TOK // 16, _scat, 0)

        pltpu.sync_copy(src_v, src_sh)

        @pl.when(c == 0)
        def _emit():
            pltpu.sync_copy(dest_v, dest_hbm)
            pltpu.sync_copy(gsort_v, gsort_hbm)
            pltpu.sync_copy(be_v, be_hbm)

    plsc.subcore_barrier()

    wid = s * NC + c
    bpw = NP // NW
    half = bpw // 2
    base = wid * bpw
    pltpu.sync_copy(src_sh.at[pl.ds(base, half)], idx_v)
    cp_a = pltpu.async_copy(x_hbm.at[idx_v], rows_v, sem)
    pltpu.sync_copy(src_sh.at[pl.ds(base + half, half)], idx_b)
    cp_b = pltpu.async_copy(x_hbm.at[idx_b], rows_b, sem_b)
    cp_a.wait()
    pltpu.sync_copy(rows_v, xs_hbm.at[pl.ds(base, half)])
    cp_b.wait()
    pltpu.sync_copy(rows_b, xs_hbm.at[pl.ds(base + half, half)])


def _dispatch(eid, rank, gate, cnt, x):
    mesh = plsc.VectorSubcoreMesh(core_axis_name="c", subcore_axis_name="s")
    bpw = NP // NW
    f = pl.kernel(
        _dispatch_body,
        compiler_params=pltpu.CompilerParams(needs_layout_passes=False),
        out_type=[
            jax.ShapeDtypeStruct((N_TOK,), jnp.int32),
            jax.ShapeDtypeStruct((NP,), jnp.float32),
            jax.ShapeDtypeStruct((NB,), jnp.int32),
            jax.ShapeDtypeStruct((NP, DIM), jnp.float32),
        ],
        mesh=mesh,
        scratch_types=[
            pltpu.VMEM((N_TOK,), jnp.int32),
            pltpu.VMEM((N_TOK,), jnp.int32),
            pltpu.VMEM((N_TOK,), jnp.float32),
            pltpu.VMEM((N_EXPERT,), jnp.int32),
            pltpu.VMEM((N_EXPERT,), jnp.int32),
            pltpu.VMEM((N_TOK,), jnp.int32),
            pltpu.VMEM((NP,), jnp.int32),
            pltpu.VMEM((NP,), jnp.float32),
            pltpu.VMEM((NB,), jnp.int32),
            pltpu.VMEM_SHARED((NP,), jnp.int32),
            pltpu.VMEM((bpw // 2,), jnp.int32),
            pltpu.VMEM((bpw // 2, DIM), jnp.float32),
            pltpu.SemaphoreType.DMA,
            pltpu.VMEM((bpw // 2,), jnp.int32),
            pltpu.VMEM((bpw // 2, DIM), jnp.float32),
            pltpu.SemaphoreType.DMA,
        ],
    )
    return f(eid, rank, gate, cnt, x)


def _mlp_body(be_ref, x_ref, gs_ref, w1_ref, w3_ref, w2_ref, out_ref):
    xb = x_ref[...]
    h = jax.nn.silu(jnp.dot(xb, w1_ref[0], preferred_element_type=jnp.float32)) * jnp.dot(
        xb, w3_ref[0], preferred_element_type=jnp.float32
    )
    y = jnp.dot(h, w2_ref[0], preferred_element_type=jnp.float32)
    out_ref[...] = gs_ref[...] * y


def _mlp(be, x_sorted, gsort, W1, W3, W2):
    grid_spec = pltpu.PrefetchScalarGridSpec(
        num_scalar_prefetch=1,
        grid=(NB,),
        in_specs=[
            pl.BlockSpec((BT, DIM), lambda b, be: (b, 0)),
            pl.BlockSpec((BT, 1), lambda b, be: (b, 0)),
            pl.BlockSpec((1, DIM, HID), lambda b, be: (be[b], 0, 0)),
            pl.BlockSpec((1, DIM, HID), lambda b, be: (be[b], 0, 0)),
            pl.BlockSpec((1, HID, DIM), lambda b, be: (be[b], 0, 0)),
        ],
        out_specs=pl.BlockSpec((BT, DIM), lambda b, be: (b, 0)),
    )
    return pl.pallas_call(
        _mlp_body,
        grid_spec=grid_spec,
        out_shape=jax.ShapeDtypeStruct((NP, DIM), jnp.float32),
        compiler_params=pltpu.CompilerParams(
            vmem_limit_bytes=112 * 1024 * 1024,
        ),
    )(be, x_sorted, gsort, W1, W3, W2)


def _combine_body(y_hbm, dest_hbm, out_hbm, idx_v, rows_v, sem):
    c = lax.axis_index("c")
    s = lax.axis_index("s")
    wid = s * NC + c
    bpw = N_TOK // NW
    base = wid * bpw
    pltpu.sync_copy(dest_hbm.at[pl.ds(base, bpw)], idx_v)
    pltpu.async_copy(y_hbm.at[idx_v], rows_v, sem).wait()
    pltpu.sync_copy(rows_v, out_hbm.at[pl.ds(base, bpw)])


def _combine(y_sorted, dest):
    mesh = plsc.VectorSubcoreMesh(core_axis_name="c", subcore_axis_name="s")
    bpw = N_TOK // NW
    f = pl.kernel(
        _combine_body,
        out_type=[jax.ShapeDtypeStruct((N_TOK, DIM), jnp.float32)],
        mesh=mesh,
        scratch_types=[
            pltpu.VMEM((bpw,), jnp.int32),
            pltpu.VMEM((bpw, DIM), jnp.float32),
            pltpu.SemaphoreType.DMA,
        ],
    )
    return f(y_sorted, dest)[0]


@jax.jit
def kernel(x, Wg, W1, W3, W2):
    eid2, gate2, rank2, cnt2 = _router(x, Wg)
    dest, gsort, be, x_sorted = _dispatch(
        eid2.reshape(N_TOK), rank2.reshape(N_TOK), gate2.reshape(N_TOK),
        cnt2.reshape(N_EXPERT), x,
    )
    y_sorted = _mlp(be, x_sorted, gsort.reshape(NP, 1), W1, W3, W2)
    return _combine(y_sorted, dest)


# speedup vs baseline: 1.0024x; 1.0024x over previous
import jax
import jax.numpy as jnp
from jax import lax
from jax.experimental import pallas as pl
from jax.experimental.pallas import tpu as pltpu
from jax.experimental.pallas import tpu_sc as plsc

DIM = 768
N_EXPERT = 16
MULT = 4
HID = DIM * MULT
N_TOK = 2048

BT = 128
NB = 32
NP = NB * BT
BH = 1536
NH = HID // BH

NC = 2
NS = 16
NW = NC * NS


def _router_body(x_ref, wg_ref, eid_ref, gate_ref, rank_ref, cnt_ref):
    x = x_ref[...]
    logits = jnp.dot(x, wg_ref[...], preferred_element_type=jnp.float32)
    probs = jax.nn.softmax(logits, axis=-1)
    top = jnp.argmax(probs, axis=-1)
    gate = jnp.max(probs, axis=-1)
    lane = jax.lax.broadcasted_iota(jnp.int32, (N_TOK, N_EXPERT), 1)
    onehot = jnp.where(lane == top[:, None], 1.0, 0.0)
    ii = jax.lax.broadcasted_iota(jnp.int32, (N_TOK, N_TOK), 0)
    jj = jax.lax.broadcasted_iota(jnp.int32, (N_TOK, N_TOK), 1)
    ltri = jnp.where(ii > jj, 1.0, 0.0)
    rk = jnp.dot(ltri, onehot, preferred_element_type=jnp.float32)
    rank = jnp.sum(rk * onehot, axis=1)
    eid_ref[...] = top[:, None].astype(jnp.int32)
    gate_ref[...] = gate[:, None]
    rank_ref[...] = rank[:, None].astype(jnp.int32)
    cnt_ref[...] = jnp.sum(onehot, axis=0, keepdims=True).astype(jnp.int32)


def _router(x, Wg):
    return pl.pallas_call(
        _router_body,
        out_shape=[
            jax.ShapeDtypeStruct((N_TOK, 1), jnp.int32),
            jax.ShapeDtypeStruct((N_TOK, 1), jnp.float32),
            jax.ShapeDtypeStruct((N_TOK, 1), jnp.int32),
            jax.ShapeDtypeStruct((1, N_EXPERT), jnp.int32),
        ],
    )(x, Wg)


def _dispatch_body(eid_hbm, rank_hbm, gate_hbm, cnt_hbm, x_hbm,
                   dest_hbm, gsort_hbm, be_hbm, xs_hbm,
                   eid_v, rank_v, gate_v, cnt_v, pad_v,
                   dest_v, src_v, gsort_v, be_v,
                   src_sh, idx_v, rows_v, sem):
    c = lax.axis_index("c")
    s = lax.axis_index("s")

    @pl.when(s == 0)
    def _work():
        pltpu.sync_copy(eid_hbm, eid_v)
        pltpu.sync_copy(rank_hbm, rank_v)
        pltpu.sync_copy(gate_hbm, gate_v)
        pltpu.sync_copy(cnt_hbm, cnt_v)

        cnt = cnt_v[...]
        nblk = (cnt + (BT - 1)) >> 7
        incl = plsc.cumsum(nblk)
        excl = incl - nblk
        pad_v[...] = excl * BT
        iota16 = lax.iota(jnp.int32, 16)
        last_e = jnp.max(jnp.where(nblk > 0, iota16, 0))

        for k in range(NB // 16):
            bvec = iota16 + 16 * k
            be_raw = jnp.zeros((16,), jnp.int32)
            for e in range(N_EXPERT):
                incl_e = jnp.sum(jnp.where(iota16 == e, incl, 0))
                be_raw = be_raw + jnp.where(incl_e <= bvec, 1, 0)
            be_v[pl.ds(16 * k, 16)] = jnp.minimum(be_raw, last_e)

        zeros_f = jnp.zeros((16,), jnp.float32)

        def _zfill(i, carry):
            src_v[pl.ds(i * 16, 16)] = (iota16 + i * 16) & (N_TOK - 1)
            gsort_v[pl.ds(i * 16, 16)] = zeros_f
            return carry

        lax.fori_loop(0, NP // 16, _zfill, 0)

        def _scat(i, carry):
            base = i * 16
            ev = eid_v[pl.ds(base, 16)]
            rv = rank_v[pl.ds(base, 16)]
            gv = gate_v[pl.ds(base, 16)]
            po = plsc.load_gather(pad_v, [ev])
            dv = po + rv
            dest_v[pl.ds(base, 16)] = dv
            plsc.store_scatter(src_v, [dv], iota16 + base)
            plsc.store_scatter(gsort_v, [dv], gv)
            return carry

        lax.fori_loop(0, N_TOK // 16, _scat, 0)

        pltpu.sync_copy(src_v, src_sh)

        @pl.when(c == 0)
        def _emit():
            pltpu.sync_copy(dest_v, dest_hbm)
            pltpu.sync_copy(gsort_v, gsort_hbm)
            pltpu.sync_copy(be_v, be_hbm)

    plsc.subcore_barrier()

    wid = s * NC + c
    bpw = NP // NW
    base = wid * bpw
    pltpu.sync_copy(src_sh.at[pl.ds(base, bpw)], idx_v)
    pltpu.async_copy(x_hbm.at[idx_v], rows_v, sem).wait()
    pltpu.sync_copy(rows_v, xs_hbm.at[pl.ds(base, bpw)])


def _dispatch(eid, rank, gate, cnt, x):
    mesh = plsc.VectorSubcoreMesh(core_axis_name="c", subcore_axis_name="s")
    bpw = NP // NW
    f = pl.kernel(
        _dispatch_body,
        compiler_params=pltpu.CompilerParams(needs_layout_passes=False),
        out_type=[
            jax.ShapeDtypeStruct((N_TOK,), jnp.int32),
            jax.ShapeDtypeStruct((NP,), jnp.float32),
            jax.ShapeDtypeStruct((NB,), jnp.int32),
            jax.ShapeDtypeStruct((NP, DIM), jnp.float32),
        ],
        mesh=mesh,
        scratch_types=[
            pltpu.VMEM((N_TOK,), jnp.int32),
            pltpu.VMEM((N_TOK,), jnp.int32),
            pltpu.VMEM((N_TOK,), jnp.float32),
            pltpu.VMEM((N_EXPERT,), jnp.int32),
            pltpu.VMEM((N_EXPERT,), jnp.int32),
            pltpu.VMEM((N_TOK,), jnp.int32),
            pltpu.VMEM((NP,), jnp.int32),
            pltpu.VMEM((NP,), jnp.float32),
            pltpu.VMEM((NB,), jnp.int32),
            pltpu.VMEM_SHARED((NP,), jnp.int32),
            pltpu.VMEM((bpw,), jnp.int32),
            pltpu.VMEM((bpw, DIM), jnp.float32),
            pltpu.SemaphoreType.DMA,
        ],
    )
    return f(eid, rank, gate, cnt, x)


def _mlp_body(be_ref, x_ref, gs_ref, w1_ref, w3_ref, w2_ref, out_ref):
    xb = x_ref[...]
    h = jax.nn.silu(jnp.dot(xb, w1_ref[0], preferred_element_type=jnp.float32)) * jnp.dot(
        xb, w3_ref[0], preferred_element_type=jnp.float32
    )
    y = jnp.dot(h, w2_ref[0], preferred_element_type=jnp.float32)
    out_ref[...] = gs_ref[...] * y


def _mlp(be, x_sorted, gsort, W1, W3, W2):
    grid_spec = pltpu.PrefetchScalarGridSpec(
        num_scalar_prefetch=1,
        grid=(NB,),
        in_specs=[
            pl.BlockSpec((BT, DIM), lambda b, be: (b, 0)),
            pl.BlockSpec((BT, 1), lambda b, be: (b, 0)),
            pl.BlockSpec((1, DIM, HID), lambda b, be: (be[b], 0, 0)),
            pl.BlockSpec((1, DIM, HID), lambda b, be: (be[b], 0, 0)),
            pl.BlockSpec((1, HID, DIM), lambda b, be: (be[b], 0, 0)),
        ],
        out_specs=pl.BlockSpec((BT, DIM), lambda b, be: (b, 0)),
    )
    return pl.pallas_call(
        _mlp_body,
        grid_spec=grid_spec,
        out_shape=jax.ShapeDtypeStruct((NP, DIM), jnp.float32),
        compiler_params=pltpu.CompilerParams(
            vmem_limit_bytes=112 * 1024 * 1024,
        ),
    )(be, x_sorted, gsort, W1, W3, W2)


def _combine_body(y_hbm, dest_hbm, out_hbm, idx_v, rows_v, sem):
    c = lax.axis_index("c")
    s = lax.axis_index("s")
    wid = s * NC + c
    bpw = N_TOK // NW
    base = wid * bpw
    pltpu.sync_copy(dest_hbm.at[pl.ds(base, bpw)], idx_v)
    pltpu.async_copy(y_hbm.at[idx_v], rows_v, sem).wait()
    pltpu.sync_copy(rows_v, out_hbm.at[pl.ds(base, bpw)])


def _combine(y_sorted, dest):
    mesh = plsc.VectorSubcoreMesh(core_axis_name="c", subcore_axis_name="s")
    bpw = N_TOK // NW
    f = pl.kernel(
        _combine_body,
        out_type=[jax.ShapeDtypeStruct((N_TOK, DIM), jnp.float32)],
        mesh=mesh,
        scratch_types=[
            pltpu.VMEM((bpw,), jnp.int32),
            pltpu.VMEM((bpw, DIM), jnp.float32),
            pltpu.SemaphoreType.DMA,
        ],
    )
    return f(y_sorted, dest)[0]


@jax.jit
def kernel(x, Wg, W1, W3, W2):
    eid2, gate2, rank2, cnt2 = _router(x, Wg)
    dest, gsort, be, x_sorted = _dispatch(
        eid2.reshape(N_TOK), rank2.reshape(N_TOK), gate2.reshape(N_TOK),
        cnt2.reshape(N_EXPERT), x,
    )
    y_sorted = _mlp(be, x_sorted, gsort.reshape(NP, 1), W1, W3, W2)
    return _combine(y_sorted, dest)
